# baseline (speedup 1.0000x reference)
import functools

import jax
import jax.numpy as jnp
from jax import lax
from jax.experimental import pallas as pl
from jax.experimental.pallas import tpu as pltpu
from jax.experimental.pallas import tpu_sc as plsc

N = 100000
C = 128
NB = 16
NPOOL = 12500

TILE = 2000
GRID = N // TILE

NW = 32
CHUNK = 64
CPW = 50
NP_PAD = NW * CPW * CHUNK
RPAD = 12544
RSUB = RPAD // 16

_bf = jnp.bfloat16


def _silu(v):
    return v * (0.5 * jnp.tanh(0.5 * v) + 0.5)


def _fused_body(x_ref, b_ref, t_ref, Wt_ref, bt_ref, ones_ref, g1_ref,
                be1_ref, W1_ref, b1_ref, g2_ref, be2_ref, W2_ref, b2_ref,
                Wd_ref, bd_ref, y_ref, x_cache, h_cache, acc1, acc2,
                tproj_s, A1_s, B1_s, A2_s, B2_s):
    p = pl.program_id(0)
    i = pl.program_id(1)
    rows = pl.ds(i * TILE, TILE)

    @pl.when(jnp.logical_and(p == 0, i == 0))
    def _init0():
        acc1[...] = jnp.zeros_like(acc1)
        acc2[...] = jnp.zeros_like(acc2)
        tt = t_ref[...]
        tproj_s[...] = (jnp.dot(_silu(tt).astype(_bf),
                                Wt_ref[...].astype(_bf),
                                preferred_element_type=jnp.float32)
                        + bt_ref[...])

    @pl.when(p == 0)
    def _phase0():
        xb = x_ref[...]
        z = jnp.concatenate([xb, xb * xb], axis=1).astype(_bf)
        acc1[...] += jnp.dot(ones_ref[...], z,
                             preferred_element_type=jnp.float32)
        x_cache[rows, :] = xb.astype(_bf)

    @pl.when(jnp.logical_and(p == 1, i == 0))
    def _init1():
        s = acc1[...]
        mu = s[0:1, :C] / N
        var = s[0:1, C:] / N - mu * mu
        r = jax.lax.rsqrt(var + 1e-5)
        A1_s[...] = r * g1_ref[...]
        B1_s[...] = be1_ref[...] - mu * r * g1_ref[...]

    @pl.when(p == 1)
    def _phase1():
        xb = x_cache[rows, :].astype(jnp.float32)
        a = xb * A1_s[...] + B1_s[...]
        a = _silu(a)
        h1 = jnp.dot(a.astype(_bf), W1_ref[...].astype(_bf),
                     preferred_element_type=jnp.float32) + b1_ref[...]
        bidx = b_ref[0, 0, :]
        onehot = (bidx[:, None]
                  == lax.broadcasted_iota(jnp.int32, (TILE, NB), 1)).astype(_bf)
        film = jnp.dot(onehot, tproj_s[...].astype(_bf),
                       preferred_element_type=jnp.float32)
        h = (1.0 + film[:, :C]) * h1 + film[:, C:]
        z = jnp.concatenate([h, h * h], axis=1).astype(_bf)
        acc2[...] += jnp.dot(ones_ref[...], z,
                             preferred_element_type=jnp.float32)
        h_cache[rows, :] = h.astype(_bf)

    @pl.when(jnp.logical_and(p == 2, i == 0))
    def _init2():
        s = acc2[...]
        mu = s[0:1, :C] / N
        var = s[0:1, C:] / N - mu * mu
        r = jax.lax.rsqrt(var + 1e-5)
        A2_s[...] = r * g2_ref[...]
        B2_s[...] = be2_ref[...] - mu * r * g2_ref[...]

    @pl.when(p == 2)
    def _phase2():
        h = h_cache[rows, :].astype(jnp.float32)
        a = h * A2_s[...] + B2_s[...]
        a = _silu(a)
        h2 = jnp.dot(a.astype(_bf), W2_ref[...].astype(_bf),
                     preferred_element_type=jnp.float32) + b2_ref[...]
        hres = h2 + x_cache[rows, :].astype(jnp.float32)
        y = (jnp.dot(hres.astype(_bf), Wd_ref[...].astype(_bf),
                     preferred_element_type=jnp.float32) + bd_ref[...])
        y_ref[...] = y


def _sc_scatter_body(y_hbm, ids_hbm, zeros_hbm, out_hbm, idx_v, d0, d1,
                     acc_sh, g0, g1, s0, s1):
    cid = lax.axis_index("c")
    sid = lax.axis_index("s")
    wid = sid * 2 + cid

    pltpu.sync_copy(zeros_hbm.at[pl.ds(sid * RSUB, RSUB)],
                    acc_sh.at[pl.ds(sid * RSUB, RSUB)])
    plsc.subcore_barrier()

    pltpu.sync_copy(ids_hbm.at[wid], idx_v)
    base = wid * CPW * CHUNK

    def chunk(c):
        return y_hbm.at[pl.ds(base + c * CHUNK, CHUNK)]

    def scat(c, buf, sem):
        return pltpu.make_async_copy(buf, acc_sh.at[idx_v.at[c]], sem)

    pltpu.async_copy(chunk(0), d0, g0)
    pltpu.async_copy(chunk(1), d1, g1)

    def body(i, carry):
        c0 = 2 * i
        pltpu.make_async_copy(chunk(c0), d0, g0).wait()
        pltpu.async_copy(d0, acc_sh.at[idx_v.at[c0]], s0, add=True)
        pltpu.make_async_copy(chunk(c0 + 1), d1, g1).wait()
        pltpu.async_copy(d1, acc_sh.at[idx_v.at[c0 + 1]], s1, add=True)
        scat(c0, d0, s0).wait()
        pltpu.async_copy(chunk(c0 + 2), d0, g0)
        scat(c0 + 1, d1, s1).wait()
        pltpu.async_copy(chunk(c0 + 3), d1, g1)
        return carry

    lax.fori_loop(0, CPW // 2 - 1, body, 0)
    pltpu.make_async_copy(chunk(CPW - 2), d0, g0).wait()
    pltpu.async_copy(d0, acc_sh.at[idx_v.at[CPW - 2]], s0, add=True)
    pltpu.make_async_copy(chunk(CPW - 1), d1, g1).wait()
    pltpu.async_copy(d1, acc_sh.at[idx_v.at[CPW - 1]], s1, add=True)
    scat(CPW - 2, d0, s0).wait()
    scat(CPW - 1, d1, s1).wait()

    plsc.subcore_barrier()
    pltpu.sync_copy(acc_sh.at[pl.ds(sid * RSUB, RSUB)],
                    out_hbm.at[pl.ds(cid * RPAD + sid * RSUB, RSUB)])


def _combine_body(p_ref, out_ref):
    a = p_ref[0, :NPOOL, :].astype(jnp.float32)
    b = p_ref[1, :NPOOL, :].astype(jnp.float32)
    out_ref[...] = a + b


def kernel(x, t, b, pool_ids, g1, be1, W1, b1, Wt, bt, g2, be2, W2, b2, Wd, bd):
    f32 = jnp.float32
    b_i = b.astype(jnp.int32).reshape(GRID, 1, TILE)
    ids = pool_ids.astype(jnp.int32)
    ids_pad = jnp.concatenate(
        [ids, jnp.full((NP_PAD - N,), NPOOL, jnp.int32)]).reshape(NW, CPW, CHUNK)
    g1r, be1r, b1r = g1.reshape(1, C), be1.reshape(1, C), b1.reshape(1, C)
    g2r, be2r, b2r = g2.reshape(1, C), be2.reshape(1, C), b2.reshape(1, C)
    bdr = bd.reshape(1, C)
    btr = bt.reshape(1, 2 * C)
    ones8 = jnp.ones((8, TILE), _bf)

    xspec = pl.BlockSpec((TILE, C), lambda p, i: (jnp.where(p == 0, i, 0), 0))
    bspec = pl.BlockSpec((1, 1, TILE), lambda p, i: (i, 0, 0))
    full = lambda shp: pl.BlockSpec(shp, lambda p, i: tuple(0 for _ in shp))

    y = pl.pallas_call(
        _fused_body,
        grid=(3, GRID),
        in_specs=[xspec, bspec, full((NB, C)), full((C, 2 * C)),
                  full((1, 2 * C)), full((8, TILE)), full((1, C)),
                  full((1, C)), full((C, C)), full((1, C)), full((1, C)),
                  full((1, C)), full((C, C)), full((1, C)), full((C, C)),
                  full((1, C))],
        out_specs=pl.BlockSpec((TILE, C),
                               lambda p, i: (jnp.where(p == 2, i, 0), 0)),
        out_shape=jax.ShapeDtypeStruct((NP_PAD, C), f32),
        scratch_shapes=[
            pltpu.VMEM((N, C), _bf),
            pltpu.VMEM((N, C), _bf),
            pltpu.VMEM((8, 2 * C), f32),
            pltpu.VMEM((8, 2 * C), f32),
            pltpu.VMEM((NB, 2 * C), f32),
            pltpu.VMEM((1, C), f32), pltpu.VMEM((1, C), f32),
            pltpu.VMEM((1, C), f32), pltpu.VMEM((1, C), f32),
        ],
    )(x, b_i, t, Wt, btr, ones8, g1r, be1r, W1, b1r, g2r, be2r, W2, b2r,
      Wd, bdr)

    zeros_hbm = jnp.zeros((RPAD, C), f32)
    mesh = plsc.VectorSubcoreMesh(core_axis_name="c", subcore_axis_name="s")
    partials = pl.kernel(
        _sc_scatter_body,
        mesh=mesh,
        out_type=jax.ShapeDtypeStruct((2 * RPAD, C), f32),
        scratch_types=[
            pltpu.VMEM((CPW, CHUNK), jnp.int32),
            pltpu.VMEM((CHUNK, C), f32),
            pltpu.VMEM((CHUNK, C), f32),
            pltpu.VMEM_SHARED((RPAD, C), f32),
            pltpu.SemaphoreType.DMA,
            pltpu.SemaphoreType.DMA,
            pltpu.SemaphoreType.DMA,
            pltpu.SemaphoreType.DMA,
        ],
    )(y, ids_pad, zeros_hbm)

    out = pl.pallas_call(
        _combine_body,
        grid=(1,),
        in_specs=[pl.BlockSpec((2, RPAD, C), lambda i: (0, 0, 0))],
        out_specs=pl.BlockSpec((NPOOL, C), lambda i: (0, 0)),
        out_shape=jax.ShapeDtypeStruct((NPOOL, C), f32),
    )(partials.reshape(2, RPAD, C))

    return out


# speedup vs baseline: 1.0725x; 1.0725x over previous
import functools

import jax
import jax.numpy as jnp
from jax import lax
from jax.experimental import pallas as pl
from jax.experimental.pallas import tpu as pltpu
from jax.experimental.pallas import tpu_sc as plsc

N = 100000
C = 128
NB = 16
NPOOL = 12500

TILE = 2000
GRID = N // TILE

NW = 32
CHUNK = 96
CPW = 33
NP_PAD = NW * CPW * CHUNK
RPAD = 12544
RSUB = RPAD // 16

_bf = jnp.bfloat16


def _silu(v):
    return v * (0.5 * jnp.tanh(0.5 * v) + 0.5)


def _fused_body(x_ref, b_ref, t_ref, Wt_ref, bt_ref, ones_ref, g1_ref,
                be1_ref, W1_ref, b1_ref, g2_ref, be2_ref, W2_ref, b2_ref,
                Wd_ref, bd_ref, y_ref, x_cache, h_cache, acc1, acc2,
                tproj_s, A1_s, B1_s, A2_s, B2_s):
    p = pl.program_id(0)
    i = pl.program_id(1)
    rows = pl.ds(i * TILE, TILE)

    @pl.when(jnp.logical_and(p == 0, i == 0))
    def _init0():
        acc1[...] = jnp.zeros_like(acc1)
        acc2[...] = jnp.zeros_like(acc2)
        tt = t_ref[...]
        tproj_s[...] = (jnp.dot(_silu(tt).astype(_bf),
                                Wt_ref[...].astype(_bf),
                                preferred_element_type=jnp.float32)
                        + bt_ref[...])

    @pl.when(p == 0)
    def _phase0():
        xb = x_ref[...]
        z = jnp.concatenate([xb, xb * xb], axis=1).astype(_bf)
        acc1[...] += jnp.dot(ones_ref[...], z,
                             preferred_element_type=jnp.float32)
        x_cache[rows, :] = xb.astype(_bf)

    @pl.when(jnp.logical_and(p == 1, i == 0))
    def _init1():
        s = acc1[...]
        mu = s[0:1, :C] / N
        var = s[0:1, C:] / N - mu * mu
        r = jax.lax.rsqrt(var + 1e-5)
        A1_s[...] = r * g1_ref[...]
        B1_s[...] = be1_ref[...] - mu * r * g1_ref[...]

    @pl.when(p == 1)
    def _phase1():
        xb = x_cache[rows, :].astype(jnp.float32)
        a = xb * A1_s[...] + B1_s[...]
        a = _silu(a)
        h1 = jnp.dot(a.astype(_bf), W1_ref[...].astype(_bf),
                     preferred_element_type=jnp.float32) + b1_ref[...]
        bidx = b_ref[0, 0, :]
        onehot = (bidx[:, None]
                  == lax.broadcasted_iota(jnp.int32, (TILE, NB), 1)).astype(_bf)
        film = jnp.dot(onehot, tproj_s[...].astype(_bf),
                       preferred_element_type=jnp.float32)
        h = (1.0 + film[:, :C]) * h1 + film[:, C:]
        z = jnp.concatenate([h, h * h], axis=1).astype(_bf)
        acc2[...] += jnp.dot(ones_ref[...], z,
                             preferred_element_type=jnp.float32)
        h_cache[rows, :] = h.astype(_bf)

    @pl.when(jnp.logical_and(p == 2, i == 0))
    def _init2():
        s = acc2[...]
        mu = s[0:1, :C] / N
        var = s[0:1, C:] / N - mu * mu
        r = jax.lax.rsqrt(var + 1e-5)
        A2_s[...] = r * g2_ref[...]
        B2_s[...] = be2_ref[...] - mu * r * g2_ref[...]

    @pl.when(p == 2)
    def _phase2():
        h = h_cache[rows, :].astype(jnp.float32)
        a = h * A2_s[...] + B2_s[...]
        a = _silu(a)
        h2 = jnp.dot(a.astype(_bf), W2_ref[...].astype(_bf),
                     preferred_element_type=jnp.float32) + b2_ref[...]
        hres = h2 + x_cache[rows, :].astype(jnp.float32)
        y = (jnp.dot(hres.astype(_bf), Wd_ref[...].astype(_bf),
                     preferred_element_type=jnp.float32) + bd_ref[...])
        y_ref[...] = y


def _sc_scatter_body(y_hbm, ids_hbm, zeros_hbm, out_hbm, idx_v, d0, d1,
                     acc_sh, g0, g1):
    cid = lax.axis_index("c")
    sid = lax.axis_index("s")
    wid = sid * 2 + cid

    pltpu.sync_copy(zeros_hbm.at[pl.ds(sid * RSUB, RSUB)],
                    acc_sh.at[pl.ds(sid * RSUB, RSUB)])
    plsc.subcore_barrier()

    pltpu.sync_copy(ids_hbm.at[wid], idx_v)
    base = wid * CPW * CHUNK

    def chunk(c):
        return y_hbm.at[pl.ds(base + c * CHUNK, CHUNK)]

    pltpu.async_copy(chunk(0), d0, g0)

    def body(i, carry):
        c0 = 2 * i
        pltpu.async_copy(chunk(c0 + 1), d1, g1)
        pltpu.make_async_copy(chunk(c0), d0, g0).wait()
        pltpu.sync_copy(d0, acc_sh.at[idx_v.at[c0]], add=True)
        pltpu.async_copy(chunk(c0 + 2), d0, g0)
        pltpu.make_async_copy(chunk(c0 + 1), d1, g1).wait()
        pltpu.sync_copy(d1, acc_sh.at[idx_v.at[c0 + 1]], add=True)
        return carry

    lax.fori_loop(0, CPW // 2, body, 0)
    pltpu.make_async_copy(chunk(CPW - 1), d0, g0).wait()
    pltpu.sync_copy(d0, acc_sh.at[idx_v.at[CPW - 1]], add=True)

    plsc.subcore_barrier()
    pltpu.sync_copy(acc_sh.at[pl.ds(sid * RSUB, RSUB)],
                    out_hbm.at[pl.ds(cid * RPAD + sid * RSUB, RSUB)])


def _combine_body(p_ref, out_ref):
    a = p_ref[0, :NPOOL, :].astype(jnp.float32)
    b = p_ref[1, :NPOOL, :].astype(jnp.float32)
    out_ref[...] = a + b


def kernel(x, t, b, pool_ids, g1, be1, W1, b1, Wt, bt, g2, be2, W2, b2, Wd, bd):
    f32 = jnp.float32
    b_i = b.astype(jnp.int32).reshape(GRID, 1, TILE)
    ids = pool_ids.astype(jnp.int32)
    ids_pad = jnp.concatenate(
        [ids, jnp.full((NP_PAD - N,), NPOOL, jnp.int32)]).reshape(NW, CPW, CHUNK)
    g1r, be1r, b1r = g1.reshape(1, C), be1.reshape(1, C), b1.reshape(1, C)
    g2r, be2r, b2r = g2.reshape(1, C), be2.reshape(1, C), b2.reshape(1, C)
    bdr = bd.reshape(1, C)
    btr = bt.reshape(1, 2 * C)
    ones8 = jnp.ones((8, TILE), _bf)

    xspec = pl.BlockSpec((TILE, C), lambda p, i: (jnp.where(p == 0, i, 0), 0))
    bspec = pl.BlockSpec((1, 1, TILE), lambda p, i: (i, 0, 0))
    full = lambda shp: pl.BlockSpec(shp, lambda p, i: tuple(0 for _ in shp))

    y = pl.pallas_call(
        _fused_body,
        grid=(3, GRID),
        in_specs=[xspec, bspec, full((NB, C)), full((C, 2 * C)),
                  full((1, 2 * C)), full((8, TILE)), full((1, C)),
                  full((1, C)), full((C, C)), full((1, C)), full((1, C)),
                  full((1, C)), full((C, C)), full((1, C)), full((C, C)),
                  full((1, C))],
        out_specs=pl.BlockSpec((TILE, C),
                               lambda p, i: (jnp.where(p == 2, i, 0), 0)),
        out_shape=jax.ShapeDtypeStruct((NP_PAD, C), f32),
        scratch_shapes=[
            pltpu.VMEM((N, C), _bf),
            pltpu.VMEM((N, C), _bf),
            pltpu.VMEM((8, 2 * C), f32),
            pltpu.VMEM((8, 2 * C), f32),
            pltpu.VMEM((NB, 2 * C), f32),
            pltpu.VMEM((1, C), f32), pltpu.VMEM((1, C), f32),
            pltpu.VMEM((1, C), f32), pltpu.VMEM((1, C), f32),
        ],
    )(x, b_i, t, Wt, btr, ones8, g1r, be1r, W1, b1r, g2r, be2r, W2, b2r,
      Wd, bdr)

    zeros_hbm = jnp.zeros((RPAD, C), f32)
    mesh = plsc.VectorSubcoreMesh(core_axis_name="c", subcore_axis_name="s")
    partials = pl.kernel(
        _sc_scatter_body,
        mesh=mesh,
        out_type=jax.ShapeDtypeStruct((2 * RPAD, C), f32),
        scratch_types=[
            pltpu.VMEM((CPW, CHUNK), jnp.int32),
            pltpu.VMEM((CHUNK, C), f32),
            pltpu.VMEM((CHUNK, C), f32),
            pltpu.VMEM_SHARED((RPAD, C), f32),
            pltpu.SemaphoreType.DMA,
            pltpu.SemaphoreType.DMA,
        ],
    )(y, ids_pad, zeros_hbm)

    out = pl.pallas_call(
        _combine_body,
        grid=(1,),
        in_specs=[pl.BlockSpec((2, RPAD, C), lambda i: (0, 0, 0))],
        out_specs=pl.BlockSpec((NPOOL, C), lambda i: (0, 0)),
        out_shape=jax.ShapeDtypeStruct((NPOOL, C), f32),
    )(partials.reshape(2, RPAD, C))

    return out


# speedup vs baseline: 1.0907x; 1.0170x over previous
import functools

import jax
import jax.numpy as jnp
from jax import lax
from jax.experimental import pallas as pl
from jax.experimental.pallas import tpu as pltpu
from jax.experimental.pallas import tpu_sc as plsc

N = 100000
C = 128
NB = 16
NPOOL = 12500

TILE = 2000
GRID = N // TILE

NW = 32
CHUNK = 96
CPW = 33
NP_PAD = NW * CPW * CHUNK
RPAD = 12544
RSUB = RPAD // 16

_bf = jnp.bfloat16


def _silu(v):
    return v * (0.5 * jnp.tanh(0.5 * v) + 0.5)


def _fused_body(x_ref, b_ref, sums_ref, t_ref, Wt_ref, bt_ref, ones_ref,
                g1_ref, be1_ref, W1_ref, b1_ref, g2_ref, be2_ref, W2_ref,
                b2_ref, Wd_ref, bd_ref, y_ref, x_cache, h_cache, acc2,
                tproj_s, A1_s, B1_s, A2_s, B2_s):
    p = pl.program_id(0)
    i = pl.program_id(1)
    rows = pl.ds(i * TILE, TILE)

    @pl.when(jnp.logical_and(p == 0, i == 0))
    def _init0():
        acc2[...] = jnp.zeros_like(acc2)
        tt = t_ref[...]
        tproj_s[...] = (jnp.dot(_silu(tt).astype(_bf),
                                Wt_ref[...].astype(_bf),
                                preferred_element_type=jnp.float32)
                        + bt_ref[...])
        s = jnp.sum(sums_ref[...], axis=0)
        mu = s[0:1, :] / N
        var = s[1:2, :] / N - mu * mu
        r = jax.lax.rsqrt(var + 1e-5)
        A1_s[...] = r * g1_ref[...]
        B1_s[...] = be1_ref[...] - mu * r * g1_ref[...]

    @pl.when(p == 0)
    def _phase1():
        xb = x_ref[...]
        x_cache[rows, :] = xb.astype(_bf)
        a = xb * A1_s[...] + B1_s[...]
        a = _silu(a)
        h1 = jnp.dot(a.astype(_bf), W1_ref[...].astype(_bf),
                     preferred_element_type=jnp.float32) + b1_ref[...]
        bidx = b_ref[0, 0, :]
        onehot = (bidx[:, None]
                  == lax.broadcasted_iota(jnp.int32, (TILE, NB), 1)).astype(_bf)
        film = jnp.dot(onehot, tproj_s[...].astype(_bf),
                       preferred_element_type=jnp.float32)
        h = (1.0 + film[:, :C]) * h1 + film[:, C:]
        z = jnp.concatenate([h, h * h], axis=1).astype(_bf)
        acc2[...] += jnp.dot(ones_ref[...], z,
                             preferred_element_type=jnp.float32)
        h_cache[rows, :] = h.astype(_bf)

    @pl.when(jnp.logical_and(p == 1, i == 0))
    def _init2():
        s = acc2[...]
        mu = s[0:1, :C] / N
        var = s[0:1, C:] / N - mu * mu
        r = jax.lax.rsqrt(var + 1e-5)
        A2_s[...] = r * g2_ref[...]
        B2_s[...] = be2_ref[...] - mu * r * g2_ref[...]

    @pl.when(p == 1)
    def _phase2():
        h = h_cache[rows, :].astype(jnp.float32)
        a = h * A2_s[...] + B2_s[...]
        a = _silu(a)
        h2 = jnp.dot(a.astype(_bf), W2_ref[...].astype(_bf),
                     preferred_element_type=jnp.float32) + b2_ref[...]
        hres = h2 + x_cache[rows, :].astype(jnp.float32)
        y = (jnp.dot(hres.astype(_bf), Wd_ref[...].astype(_bf),
                     preferred_element_type=jnp.float32) + bd_ref[...])
        y_ref[...] = y


SCH = 80
NCH = N // SCH
SCPW = 39


def _sc_stats_body(x_hbm, out_hbm, d0, d1, stage, g0, g1):
    cid = lax.axis_index("c")
    sid = lax.axis_index("s")
    wid = sid * 2 + cid

    def chunk(c):
        return x_hbm.at[pl.ds(c * SCH, SCH)]

    def accum(buf, carry):
        def row_body(m, cr):
            ss, qq = cr
            for dr in range(2):
                r = 2 * m + dr
                ss = list(ss)
                qq = list(qq)
                for k in range(8):
                    v = buf[r, pl.ds(16 * k, 16)]
                    ss[k] = ss[k] + v
                    qq[k] = qq[k] + v * v
                ss = tuple(ss)
                qq = tuple(qq)
            return ss, qq

        return lax.fori_loop(0, SCH // 2, row_body, carry)

    zero = tuple(jnp.zeros((16,), jnp.float32) for _ in range(8))
    carry = (zero, zero)
    pltpu.async_copy(chunk(wid), d0, g0)

    def body(j, cr):
        c0 = wid + 64 * j
        pltpu.async_copy(chunk(c0 + 32), d1, g1)
        pltpu.make_async_copy(chunk(c0), d0, g0).wait()
        cr = accum(d0, cr)

        @pl.when(j + 1 < (SCPW + 1) // 2)
        def _pf():
            pltpu.async_copy(chunk(c0 + 64), d0, g0)

        pltpu.make_async_copy(chunk(c0 + 32), d1, g1).wait()
        cr = accum(d1, cr)
        return cr

    carry = lax.fori_loop(0, SCPW // 2, body, carry)
    pltpu.make_async_copy(chunk(wid + 32 * (SCPW - 1)), d0, g0).wait()
    carry = accum(d0, carry)

    c_extra = jnp.where(wid < 2, NCH - 2 + wid, 0)
    pltpu.sync_copy(x_hbm.at[pl.ds(c_extra * SCH, SCH)], d1)
    extra = accum(d1, (zero, zero))
    m = jnp.where(wid < 2, 1.0, 0.0).astype(jnp.float32)
    ss = tuple(s + m * e for s, e in zip(carry[0], extra[0]))
    qq = tuple(s + m * e for s, e in zip(carry[1], extra[1]))

    for k in range(8):
        stage[0, pl.ds(16 * k, 16)] = ss[k]
        stage[1, pl.ds(16 * k, 16)] = qq[k]
    pltpu.sync_copy(stage, out_hbm.at[wid])


def _sc_scatter_body(y_hbm, ids_hbm, zeros_hbm, out_hbm, idx_v, d0, d1,
                     acc_sh, g0, g1):
    cid = lax.axis_index("c")
    sid = lax.axis_index("s")
    wid = sid * 2 + cid

    pltpu.sync_copy(zeros_hbm.at[pl.ds(sid * RSUB, RSUB)],
                    acc_sh.at[pl.ds(sid * RSUB, RSUB)])
    plsc.subcore_barrier()

    pltpu.sync_copy(ids_hbm.at[wid], idx_v)
    base = wid * CPW * CHUNK

    def chunk(c):
        return y_hbm.at[pl.ds(base + c * CHUNK, CHUNK)]

    pltpu.async_copy(chunk(0), d0, g0)

    def body(i, carry):
        c0 = 2 * i
        pltpu.async_copy(chunk(c0 + 1), d1, g1)
        pltpu.make_async_copy(chunk(c0), d0, g0).wait()
        pltpu.sync_copy(d0, acc_sh.at[idx_v.at[c0]], add=True)
        pltpu.async_copy(chunk(c0 + 2), d0, g0)
        pltpu.make_async_copy(chunk(c0 + 1), d1, g1).wait()
        pltpu.sync_copy(d1, acc_sh.at[idx_v.at[c0 + 1]], add=True)
        return carry

    lax.fori_loop(0, CPW // 2, body, 0)
    pltpu.make_async_copy(chunk(CPW - 1), d0, g0).wait()
    pltpu.sync_copy(d0, acc_sh.at[idx_v.at[CPW - 1]], add=True)

    plsc.subcore_barrier()
    pltpu.sync_copy(acc_sh.at[pl.ds(sid * RSUB, RSUB)],
                    out_hbm.at[pl.ds(cid * RPAD + sid * RSUB, RSUB)])


def _combine_body(p_ref, out_ref):
    a = p_ref[0, :NPOOL, :].astype(jnp.float32)
    b = p_ref[1, :NPOOL, :].astype(jnp.float32)
    out_ref[...] = a + b


def kernel(x, t, b, pool_ids, g1, be1, W1, b1, Wt, bt, g2, be2, W2, b2, Wd, bd):
    f32 = jnp.float32
    b_i = b.astype(jnp.int32).reshape(GRID, 1, TILE)
    ids = pool_ids.astype(jnp.int32)
    ids_pad = jnp.concatenate(
        [ids, jnp.full((NP_PAD - N,), NPOOL, jnp.int32)]).reshape(NW, CPW, CHUNK)
    g1r, be1r, b1r = g1.reshape(1, C), be1.reshape(1, C), b1.reshape(1, C)
    g2r, be2r, b2r = g2.reshape(1, C), be2.reshape(1, C), b2.reshape(1, C)
    bdr = bd.reshape(1, C)
    btr = bt.reshape(1, 2 * C)
    ones8 = jnp.ones((8, TILE), _bf)

    mesh = plsc.VectorSubcoreMesh(core_axis_name="c", subcore_axis_name="s")
    sums32 = pl.kernel(
        _sc_stats_body,
        mesh=mesh,
        out_type=jax.ShapeDtypeStruct((NW, 2, C), f32),
        scratch_types=[
            pltpu.VMEM((SCH, C), f32),
            pltpu.VMEM((SCH, C), f32),
            pltpu.VMEM((2, C), f32),
            pltpu.SemaphoreType.DMA,
            pltpu.SemaphoreType.DMA,
        ],
    )(x)

    xspec = pl.BlockSpec((TILE, C), lambda p, i: (jnp.where(p == 0, i, 0), 0))
    bspec = pl.BlockSpec((1, 1, TILE), lambda p, i: (i, 0, 0))
    full = lambda shp: pl.BlockSpec(shp, lambda p, i: tuple(0 for _ in shp))

    y = pl.pallas_call(
        _fused_body,
        grid=(2, GRID),
        in_specs=[xspec, bspec, full((NW, 2, C)), full((NB, C)),
                  full((C, 2 * C)), full((1, 2 * C)), full((8, TILE)),
                  full((1, C)), full((1, C)), full((C, C)), full((1, C)),
                  full((1, C)), full((1, C)), full((C, C)), full((1, C)),
                  full((C, C)), full((1, C))],
        out_specs=pl.BlockSpec((TILE, C),
                               lambda p, i: (jnp.where(p == 1, i, 0), 0)),
        out_shape=jax.ShapeDtypeStruct((NP_PAD, C), f32),
        scratch_shapes=[
            pltpu.VMEM((N, C), _bf),
            pltpu.VMEM((N, C), _bf),
            pltpu.VMEM((8, 2 * C), f32),
            pltpu.VMEM((NB, 2 * C), f32),
            pltpu.VMEM((1, C), f32), pltpu.VMEM((1, C), f32),
            pltpu.VMEM((1, C), f32), pltpu.VMEM((1, C), f32),
        ],
    )(x, b_i, sums32, t, Wt, btr, ones8, g1r, be1r, W1, b1r, g2r, be2r,
      W2, b2r, Wd, bdr)

    zeros_hbm = jnp.zeros((RPAD, C), f32)
    partials = pl.kernel(
        _sc_scatter_body,
        mesh=mesh,
        out_type=jax.ShapeDtypeStruct((2 * RPAD, C), f32),
        scratch_types=[
            pltpu.VMEM((CPW, CHUNK), jnp.int32),
            pltpu.VMEM((CHUNK, C), f32),
            pltpu.VMEM((CHUNK, C), f32),
            pltpu.VMEM_SHARED((RPAD, C), f32),
            pltpu.SemaphoreType.DMA,
            pltpu.SemaphoreType.DMA,
        ],
    )(y, ids_pad, zeros_hbm)

    out = pl.pallas_call(
        _combine_body,
        grid=(1,),
        in_specs=[pl.BlockSpec((2, RPAD, C), lambda i: (0, 0, 0))],
        out_specs=pl.BlockSpec((NPOOL, C), lambda i: (0, 0)),
        out_shape=jax.ShapeDtypeStruct((NPOOL, C), f32),
    )(partials.reshape(2, RPAD, C))

    return out


# speedup vs baseline: 1.0911x; 1.0004x over previous
import jax
import jax.numpy as jnp
from jax import lax
from jax.experimental import pallas as pl
from jax.experimental.pallas import tpu as pltpu
from jax.experimental.pallas import tpu_sc as plsc

N = 100000
C = 128
NB = 16
NPOOL = 12500

TILE = 2000
GRID = N // TILE

NW = 32
CHUNK = 96
CPW = 33
NP_PAD = NW * CPW * CHUNK
RPAD = 12544
RSUB = RPAD // 16

_bf = jnp.bfloat16


def _silu(v):
    return v * (0.5 * jnp.tanh(0.5 * v) + 0.5)


def _fused_body(x_ref, b_ref, sums_ref, t_ref, Wt_ref, bt_ref, ones_ref,
                g1_ref, be1_ref, W1_ref, b1_ref, g2_ref, be2_ref, W2_ref,
                b2_ref, Wd_ref, bd_ref, y_ref, x_cache, h_cache, acc2,
                tproj_s, A1_s, B1_s, A2_s, B2_s):
    p = pl.program_id(0)
    i = pl.program_id(1)
    rows = pl.ds(i * TILE, TILE)

    @pl.when(jnp.logical_and(p == 0, i == 0))
    def _init0():
        acc2[...] = jnp.zeros_like(acc2)
        tt = t_ref[...]
        tproj_s[...] = (jnp.dot(_silu(tt).astype(_bf),
                                Wt_ref[...].astype(_bf),
                                preferred_element_type=jnp.float32)
                        + bt_ref[...])
        s = jnp.sum(sums_ref[...], axis=0)
        mu = s[0:1, :] / N
        var = s[1:2, :] / N - mu * mu
        r = jax.lax.rsqrt(var + 1e-5)
        A1_s[...] = r * g1_ref[...]
        B1_s[...] = be1_ref[...] - mu * r * g1_ref[...]

    @pl.when(p == 0)
    def _phase1():
        xb = x_ref[...]
        x_cache[rows, :] = xb.astype(_bf)
        a = xb * A1_s[...] + B1_s[...]
        a = _silu(a)
        h1 = jnp.dot(a.astype(_bf), W1_ref[...].astype(_bf),
                     preferred_element_type=jnp.float32) + b1_ref[...]
        bidx = b_ref[0, 0, :]
        onehot = (bidx[:, None]
                  == lax.broadcasted_iota(jnp.int32, (TILE, NB), 1)).astype(_bf)
        film = jnp.dot(onehot, tproj_s[...].astype(_bf),
                       preferred_element_type=jnp.float32)
        h = (1.0 + film[:, :C]) * h1 + film[:, C:]
        z = jnp.concatenate([h, h * h], axis=1).astype(_bf)
        acc2[...] += jnp.dot(ones_ref[...], z,
                             preferred_element_type=jnp.float32)
        h_cache[rows, :] = h.astype(_bf)

    @pl.when(jnp.logical_and(p == 1, i == 0))
    def _init2():
        s = acc2[...]
        mu = s[0:1, :C] / N
        var = s[0:1, C:] / N - mu * mu
        r = jax.lax.rsqrt(var + 1e-5)
        A2_s[...] = r * g2_ref[...]
        B2_s[...] = be2_ref[...] - mu * r * g2_ref[...]

    @pl.when(p == 1)
    def _phase2():
        h = h_cache[rows, :].astype(jnp.float32)
        a = h * A2_s[...] + B2_s[...]
        a = _silu(a)
        h2 = jnp.dot(a.astype(_bf), W2_ref[...].astype(_bf),
                     preferred_element_type=jnp.float32) + b2_ref[...]
        hres = h2 + x_cache[rows, :].astype(jnp.float32)
        y = (jnp.dot(hres.astype(_bf), Wd_ref[...].astype(_bf),
                     preferred_element_type=jnp.float32) + bd_ref[...])
        y_ref[...] = y


SCH = 80
NCH = N // SCH
SCPW = 39


def _sc_stats_body(x_hbm, out_hbm, d0, d1, stage, g0, g1):
    cid = lax.axis_index("c")
    sid = lax.axis_index("s")
    wid = sid * 2 + cid

    def chunk(c):
        return x_hbm.at[pl.ds(c * SCH, SCH)]

    def accum(buf, carry):
        def row_body(m, cr):
            ss, qq = cr
            for dr in range(2):
                r = 2 * m + dr
                ss = list(ss)
                qq = list(qq)
                for k in range(8):
                    v = buf[r, pl.ds(16 * k, 16)]
                    ss[k] = ss[k] + v
                    qq[k] = qq[k] + v * v
                ss = tuple(ss)
                qq = tuple(qq)
            return ss, qq

        return lax.fori_loop(0, SCH // 2, row_body, carry)

    zero = tuple(jnp.zeros((16,), jnp.float32) for _ in range(8))
    carry = (zero, zero)
    pltpu.async_copy(chunk(wid), d0, g0)

    def body(j, cr):
        c0 = wid + 64 * j
        pltpu.async_copy(chunk(c0 + 32), d1, g1)
        pltpu.make_async_copy(chunk(c0), d0, g0).wait()
        cr = accum(d0, cr)

        @pl.when(j + 1 < (SCPW + 1) // 2)
        def _pf():
            pltpu.async_copy(chunk(c0 + 64), d0, g0)

        pltpu.make_async_copy(chunk(c0 + 32), d1, g1).wait()
        cr = accum(d1, cr)
        return cr

    carry = lax.fori_loop(0, SCPW // 2, body, carry)
    pltpu.make_async_copy(chunk(wid + 32 * (SCPW - 1)), d0, g0).wait()
    carry = accum(d0, carry)

    c_extra = jnp.where(wid < 2, NCH - 2 + wid, 0)
    pltpu.sync_copy(x_hbm.at[pl.ds(c_extra * SCH, SCH)], d1)
    extra = accum(d1, (zero, zero))
    m = jnp.where(wid < 2, 1.0, 0.0).astype(jnp.float32)
    ss = tuple(s + m * e for s, e in zip(carry[0], extra[0]))
    qq = tuple(s + m * e for s, e in zip(carry[1], extra[1]))

    for k in range(8):
        stage[0, pl.ds(16 * k, 16)] = ss[k]
        stage[1, pl.ds(16 * k, 16)] = qq[k]
    pltpu.sync_copy(stage, out_hbm.at[wid])


def _sc_scatter_body(y_hbm, ids_hbm, zeros_hbm, out_hbm, idx_v, d0, d1,
                     acc_sh, g0, g1):
    cid = lax.axis_index("c")
    sid = lax.axis_index("s")
    wid = sid * 2 + cid

    pltpu.sync_copy(zeros_hbm.at[pl.ds(sid * RSUB, RSUB)],
                    acc_sh.at[pl.ds(sid * RSUB, RSUB)])
    plsc.subcore_barrier()

    pltpu.sync_copy(ids_hbm.at[wid], idx_v)
    base = wid * CPW * CHUNK

    def chunk(c):
        return y_hbm.at[pl.ds(base + c * CHUNK, CHUNK)]

    pltpu.async_copy(chunk(0), d0, g0)

    def body(i, carry):
        c0 = 2 * i
        pltpu.async_copy(chunk(c0 + 1), d1, g1)
        pltpu.make_async_copy(chunk(c0), d0, g0).wait()
        pltpu.sync_copy(d0, acc_sh.at[idx_v.at[c0]], add=True)
        pltpu.async_copy(chunk(c0 + 2), d0, g0)
        pltpu.make_async_copy(chunk(c0 + 1), d1, g1).wait()
        pltpu.sync_copy(d1, acc_sh.at[idx_v.at[c0 + 1]], add=True)
        return carry

    lax.fori_loop(0, CPW // 2, body, 0)
    pltpu.make_async_copy(chunk(CPW - 1), d0, g0).wait()
    pltpu.sync_copy(d0, acc_sh.at[idx_v.at[CPW - 1]], add=True)

    plsc.subcore_barrier()
    pltpu.sync_copy(acc_sh.at[pl.ds(sid * RSUB, RSUB)],
                    out_hbm.at[pl.ds(cid * RPAD + sid * RSUB, RSUB)])


def _combine_body(p_ref, out_ref):
    a = p_ref[0, :NPOOL, :].astype(jnp.float32)
    b = p_ref[1, :NPOOL, :].astype(jnp.float32)
    out_ref[...] = a + b


def kernel(x, t, b, pool_ids, g1, be1, W1, b1, Wt, bt, g2, be2, W2, b2, Wd, bd):
    f32 = jnp.float32
    b_i = b.astype(jnp.int32).reshape(GRID, 1, TILE)
    ids = pool_ids.astype(jnp.int32)
    ids_pad = jnp.concatenate(
        [ids, jnp.full((NP_PAD - N,), NPOOL, jnp.int32)]).reshape(NW, CPW, CHUNK)
    g1r, be1r, b1r = g1.reshape(1, C), be1.reshape(1, C), b1.reshape(1, C)
    g2r, be2r, b2r = g2.reshape(1, C), be2.reshape(1, C), b2.reshape(1, C)
    bdr = bd.reshape(1, C)
    btr = bt.reshape(1, 2 * C)
    ones8 = jnp.ones((8, TILE), _bf)

    mesh = plsc.VectorSubcoreMesh(core_axis_name="c", subcore_axis_name="s")
    sums32 = pl.kernel(
        _sc_stats_body,
        mesh=mesh,
        out_type=jax.ShapeDtypeStruct((NW, 2, C), f32),
        scratch_types=[
            pltpu.VMEM((SCH, C), f32),
            pltpu.VMEM((SCH, C), f32),
            pltpu.VMEM((2, C), f32),
            pltpu.SemaphoreType.DMA,
            pltpu.SemaphoreType.DMA,
        ],
    )(x)

    xspec = pl.BlockSpec((TILE, C), lambda p, i: (jnp.where(p == 0, i, 0), 0))
    bspec = pl.BlockSpec((1, 1, TILE), lambda p, i: (i, 0, 0))
    full = lambda shp: pl.BlockSpec(shp, lambda p, i: tuple(0 for _ in shp))

    y = pl.pallas_call(
        _fused_body,
        grid=(2, GRID),
        in_specs=[xspec, bspec, full((NW, 2, C)), full((NB, C)),
                  full((C, 2 * C)), full((1, 2 * C)), full((8, TILE)),
                  full((1, C)), full((1, C)), full((C, C)), full((1, C)),
                  full((1, C)), full((1, C)), full((C, C)), full((1, C)),
                  full((C, C)), full((1, C))],
        out_specs=pl.BlockSpec((TILE, C),
                               lambda p, i: (jnp.where(p == 1, i, 0), 0)),
        out_shape=jax.ShapeDtypeStruct((NP_PAD, C), f32),
        scratch_shapes=[
            pltpu.VMEM((N, C), _bf),
            pltpu.VMEM((N, C), _bf),
            pltpu.VMEM((8, 2 * C), f32),
            pltpu.VMEM((NB, 2 * C), f32),
            pltpu.VMEM((1, C), f32), pltpu.VMEM((1, C), f32),
            pltpu.VMEM((1, C), f32), pltpu.VMEM((1, C), f32),
        ],
    )(x, b_i, sums32, t, Wt, btr, ones8, g1r, be1r, W1, b1r, g2r, be2r,
      W2, b2r, Wd, bdr)

    zeros_hbm = jnp.zeros((RPAD, C), f32)
    partials = pl.kernel(
        _sc_scatter_body,
        mesh=mesh,
        out_type=jax.ShapeDtypeStruct((2 * RPAD, C), f32),
        scratch_types=[
            pltpu.VMEM((CPW, CHUNK), jnp.int32),
            pltpu.VMEM((CHUNK, C), f32),
            pltpu.VMEM((CHUNK, C), f32),
            pltpu.VMEM_SHARED((RPAD, C), f32),
            pltpu.SemaphoreType.DMA,
            pltpu.SemaphoreType.DMA,
        ],
    )(y, ids_pad, zeros_hbm)

    out = pl.pallas_call(
        _combine_body,
        grid=(1,),
        in_specs=[pl.BlockSpec((2, RPAD, C), lambda i: (0, 0, 0))],
        out_specs=pl.BlockSpec((NPOOL, C), lambda i: (0, 0)),
        out_shape=jax.ShapeDtypeStruct((NPOOL, C), f32),
    )(partials.reshape(2, RPAD, C))

    return out
